# baseline (speedup 1.0000x reference)
import jax
import jax.numpy as jnp
from jax.experimental import pallas as pl
from jax.experimental.pallas import tpu as pltpu
from jax.experimental.pallas import tpu_sc as plsc

D_MODEL = 64
SCALE = 8.0
IDX_ROW = 128
GATHERS = 4
WINDOW = IDX_ROW * GATHERS
LANES = 16


def _sc_embed(idx2d, lut, n):
    vector_mesh = plsc.VectorSubcoreMesh(
        core_axis_name="core", subcore_axis_name="subcore"
    )

    @pl.kernel(
        out_type=jax.ShapeDtypeStruct((n, D_MODEL), lut.dtype),
        mesh=vector_mesh,
        scratch_types=[pltpu.SemaphoreType.DMA],
        compiler_params=pltpu.CompilerParams(use_tc_tiling_on_sc=False),
    )
    def run(lut_hbm, i_hbm, o_hbm, sem):
        def body(i_vmem, o_vmem):
            copies = [
                pltpu.async_copy(
                    lut_hbm.at[i_vmem.at[j]],
                    o_vmem.at[pl.ds(j * IDX_ROW, IDX_ROW)],
                    sem,
                )
                for j in range(GATHERS)
            ]
            for c in copies:
                c.wait()

            @pl.loop(0, WINDOW)
            def _(r):
                for c in range(D_MODEL // LANES):
                    slc = (pl.ds(r, 1), pl.ds(c * LANES, LANES))
                    o_vmem.at[*slc][...] = o_vmem.at[*slc][...] * SCALE

        pltpu.emit_pipeline(
            body,
            grid=(n // WINDOW,),
            in_specs=[
                pl.BlockSpec((GATHERS, IDX_ROW), lambda i: (i, 0)),
            ],
            out_specs=[
                pl.BlockSpec((WINDOW, D_MODEL), lambda i: (i, 0)),
            ],
            core_axis_name=("core", "subcore"),
            dimension_semantics=(pltpu.PARALLEL,),
        )(i_hbm, o_hbm)

    return run(lut, idx2d)


def kernel(x, lut):
    b, s = x.shape
    n = b * s
    idx2d = x.reshape(n // IDX_ROW, IDX_ROW).astype(jnp.int32)
    out = _sc_embed(idx2d, lut, n)
    return out.reshape(b, s, D_MODEL)


# speedup vs baseline: 1.4344x; 1.4344x over previous
import jax
import jax.numpy as jnp
from jax.experimental import pallas as pl
from jax.experimental.pallas import tpu as pltpu
from jax.experimental.pallas import tpu_sc as plsc

VOCAB = 1000000
D_MODEL = 64
SCALE = 8.0
IDX_ROW = 128
GATHERS = 2
WINDOW = IDX_ROW * GATHERS
ROW_W = 128


LUT_LANES = 6400


def _pack_scale_lut(lut_t):
    grid = (VOCAB + LUT_LANES - 1) // LUT_LANES

    def body(x_ref, o_ref):
        o_ref[:, :D_MODEL] = (x_ref[...] * SCALE).T

    return pl.pallas_call(
        body,
        grid=(grid,),
        in_specs=[pl.BlockSpec((D_MODEL, LUT_LANES), lambda i: (0, i))],
        out_specs=pl.BlockSpec((LUT_LANES, ROW_W), lambda i: (i, 0)),
        out_shape=jax.ShapeDtypeStruct((VOCAB, ROW_W), jnp.float32),
    )(lut_t)


def _sc_gather(lut_lin, idx2d, n):
    vector_mesh = plsc.VectorSubcoreMesh(
        core_axis_name="core", subcore_axis_name="subcore"
    )

    @pl.kernel(
        out_type=jax.ShapeDtypeStruct((n, ROW_W), jnp.float32),
        mesh=vector_mesh,
        scratch_types=[pltpu.SemaphoreType.DMA],
        compiler_params=pltpu.CompilerParams(use_tc_tiling_on_sc=False),
    )
    def run(lut_hbm, i_hbm, o_hbm, sem):
        def body(i_vmem, o_vmem):
            copies = [
                pltpu.async_copy(
                    lut_hbm.at[i_vmem.at[j]],
                    o_vmem.at[pl.ds(j * IDX_ROW, IDX_ROW)],
                    sem,
                )
                for j in range(GATHERS)
            ]
            for c in copies:
                c.wait()

        pltpu.emit_pipeline(
            body,
            grid=(n // WINDOW,),
            in_specs=[pl.BlockSpec((GATHERS, IDX_ROW), lambda i: (i, 0))],
            out_specs=[pl.BlockSpec((WINDOW, ROW_W), lambda i: (i, 0))],
            core_axis_name=("core", "subcore"),
            dimension_semantics=(pltpu.PARALLEL,),
        )(i_hbm, o_hbm)

    return run(lut_lin, idx2d)


B_BLOCK = 128
S_BLOCK = 8


def _out_transpose(flat3d):
    b, s, _ = flat3d.shape

    def body(x_ref, o_ref):
        for k in range(S_BLOCK):
            o_ref[k] = x_ref[:, k, :D_MODEL].T

    return pl.pallas_call(
        body,
        grid=(b // B_BLOCK, s // S_BLOCK),
        in_specs=[
            pl.BlockSpec((B_BLOCK, S_BLOCK, ROW_W), lambda i, j: (i, j, 0))
        ],
        out_specs=pl.BlockSpec(
            (S_BLOCK, D_MODEL, B_BLOCK), lambda i, j: (j, 0, i)
        ),
        out_shape=jax.ShapeDtypeStruct((s, D_MODEL, b), jnp.float32),
    )(flat3d)


def kernel(x, lut):
    b, s = x.shape
    n = b * s
    packed = _pack_scale_lut(lut.T)
    idx2d = x.reshape(n // IDX_ROW, IDX_ROW).astype(jnp.int32)
    flat = _sc_gather(packed, idx2d, n)
    t2 = _out_transpose(flat.reshape(b, s, ROW_W))
    return jnp.transpose(t2, (2, 0, 1))
